# Initial kernel scaffold; baseline (speedup 1.0000x reference)
import functools

import numpy as np
import jax
import jax.numpy as jnp
from jax import lax
from jax.experimental import pallas as pl
from jax.experimental.pallas import tpu as pltpu
from jax.experimental.pallas import tpu_sc as plsc

N = 10000
E = 320000
D = 200
L = 5
G = 100

NC, NS = 2, 16
NW = NC * NS
NP = 10048
DP = 208
RPT = NP // NS
EW = E // NW
K = 128
CW = (EW + K - 1) // K
NB = 8
BR = NP // NB

_f32 = jnp.float32
_i32 = jnp.int32


def _sc_mesh():
    return plsc.VectorSubcoreMesh(core_axis_name="c", subcore_axis_name="s")


def _sc_scatter(h_pad, srcw, dstw, zrows):
    @functools.partial(
        pl.kernel,
        out_type=jax.ShapeDtypeStruct((NC, NP, DP), _f32),
        mesh=_sc_mesh(),
        scratch_types=[
            pltpu.VMEM((CW, K), _i32),
            pltpu.VMEM((CW, K), _i32),
            pltpu.VMEM((K, DP), _f32),
            pltpu.VMEM_SHARED((NP, DP), _f32),
            pltpu.SemaphoreType.DMA,
        ],
    )
    def k(h_hbm, src_hbm, dst_hbm, z_hbm, out_hbm, src_v, dst_v, buf, agg, sem):
        cid = lax.axis_index("c")
        sid = lax.axis_index("s")
        wid = cid * NS + sid
        pltpu.sync_copy(src_hbm.at[wid], src_v)
        pltpu.sync_copy(dst_hbm.at[wid], dst_v)
        r0 = sid * RPT
        pltpu.sync_copy(z_hbm, agg.at[pl.ds(r0, RPT)])
        plsc.subcore_barrier()

        @pl.loop(0, CW)
        def _(c):
            pltpu.async_copy(h_hbm.at[src_v.at[c]], buf, sem).wait()
            pltpu.sync_copy(buf, agg.at[dst_v.at[c]], add=True)

        plsc.subcore_barrier()
        pltpu.sync_copy(agg.at[pl.ds(r0, RPT)], out_hbm.at[cid, pl.ds(r0, RPT)])

    return k(h_pad, srcw, dstw, zrows)


def _sc_hist(itab, aw, dstw, z16):
    @functools.partial(
        pl.kernel,
        out_type=jax.ShapeDtypeStruct((NC, NP, 16), _f32),
        mesh=_sc_mesh(),
        scratch_types=[
            pltpu.VMEM((CW, K), _i32),
            pltpu.VMEM((CW, K), _i32),
            pltpu.VMEM((K, 16), _f32),
            pltpu.VMEM_SHARED((NP, 16), _f32),
            pltpu.SemaphoreType.DMA,
        ],
    )
    def k(t_hbm, a_hbm, dst_hbm, z_hbm, out_hbm, a_v, dst_v, buf, acc, sem):
        cid = lax.axis_index("c")
        sid = lax.axis_index("s")
        wid = cid * NS + sid
        pltpu.sync_copy(a_hbm.at[wid], a_v)
        pltpu.sync_copy(dst_hbm.at[wid], dst_v)
        r0 = sid * RPT
        pltpu.sync_copy(z_hbm, acc.at[pl.ds(r0, RPT)])
        plsc.subcore_barrier()

        @pl.loop(0, CW)
        def _(c):
            pltpu.async_copy(t_hbm.at[a_v.at[c]], buf, sem).wait()
            pltpu.sync_copy(buf, acc.at[dst_v.at[c]], add=True)

        plsc.subcore_barrier()
        pltpu.sync_copy(acc.at[pl.ds(r0, RPT)], out_hbm.at[cid, pl.ds(r0, RPT)])

    return k(itab, aw, dstw, z16)


def _tc_embed(x0, x1, emb1p, emb2p):
    def body(x0_ref, x1_ref, e1_ref, e2_ref, o_ref):
        ar = lax.broadcasted_iota(_i32, (BR, 512), 1)
        m1 = (ar == x0_ref[...]).astype(_f32)
        m2 = (ar == x1_ref[...]).astype(_f32)
        o_ref[...] = (jnp.dot(m1, e1_ref[...], preferred_element_type=_f32)
                      + jnp.dot(m2, e2_ref[...], preferred_element_type=_f32))

    return pl.pallas_call(
        body,
        grid=(NB,),
        in_specs=[
            pl.BlockSpec((BR, 1), lambda i: (i, 0)),
            pl.BlockSpec((BR, 1), lambda i: (i, 0)),
            pl.BlockSpec((512, DP), lambda i: (0, 0)),
            pl.BlockSpec((512, DP), lambda i: (0, 0)),
        ],
        out_specs=pl.BlockSpec((BR, DP), lambda i: (i, 0)),
        out_shape=jax.ShapeDtypeStruct((NP, DP), _f32),
    )(x0, x1, emb1p, emb2p)


def _tc_layer(part, h, c2, q, selfc, w1, b1, w2, b2):
    def body(p_ref, h_ref, c_ref, q_ref, sc_ref, w1_ref, b1_ref, w2_ref,
             b2_ref, u_ref, s1_ref, s2_ref):
        i = pl.program_id(0)
        agg = p_ref[0][:, :D] + p_ref[1][:, :D] + h_ref[:, :D]
        cc = c_ref[0] + c_ref[1]
        agg = agg + jnp.dot(cc, q_ref[...], preferred_element_type=_f32)
        agg = agg + sc_ref[...]
        t = jnp.maximum(
            jnp.dot(agg, w1_ref[...], preferred_element_type=_f32) + b1_ref[...],
            0.0)
        u = jnp.dot(t, w2_ref[...], preferred_element_type=_f32) + b2_ref[...]
        rows = lax.broadcasted_iota(_i32, (BR, 1), 0) + i * BR
        u = jnp.where(rows < N, u, 0.0)
        u_ref[...] = u
        su = jnp.sum(u, axis=0, keepdims=True)
        sq = jnp.sum(u * u, axis=0, keepdims=True)

        @pl.when(i == 0)
        def _():
            s1_ref[...] = su
            s2_ref[...] = sq

        @pl.when(i > 0)
        def _():
            s1_ref[...] += su
            s2_ref[...] += sq

    return pl.pallas_call(
        body,
        grid=(NB,),
        in_specs=[
            pl.BlockSpec((NC, BR, DP), lambda i: (0, i, 0)),
            pl.BlockSpec((BR, DP), lambda i: (i, 0)),
            pl.BlockSpec((NC, BR, 16), lambda i: (0, i, 0)),
            pl.BlockSpec((16, D), lambda i: (0, 0)),
            pl.BlockSpec((1, D), lambda i: (0, 0)),
            pl.BlockSpec((D, 2 * D), lambda i: (0, 0)),
            pl.BlockSpec((1, 2 * D), lambda i: (0, 0)),
            pl.BlockSpec((2 * D, D), lambda i: (0, 0)),
            pl.BlockSpec((1, D), lambda i: (0, 0)),
        ],
        out_specs=[
            pl.BlockSpec((BR, D), lambda i: (i, 0)),
            pl.BlockSpec((1, D), lambda i: (0, 0)),
            pl.BlockSpec((1, D), lambda i: (0, 0)),
        ],
        out_shape=[
            jax.ShapeDtypeStruct((NP, D), _f32),
            jax.ShapeDtypeStruct((1, D), _f32),
            jax.ShapeDtypeStruct((1, D), _f32),
        ],
    )(part, h, c2, q, selfc, w1, b1, w2, b2)


def _tc_norm(u, s1, s2, g, b, relu):
    def body(u_ref, s1_ref, s2_ref, g_ref, b_ref, o_ref):
        mean = s1_ref[...] * (1.0 / N)
        var = s2_ref[...] * (1.0 / N) - mean * mean
        inv = lax.rsqrt(var + 1e-5)
        hn = (u_ref[...] - mean) * (inv * g_ref[...]) + b_ref[...]
        if relu:
            hn = jnp.maximum(hn, 0.0)
        o_ref[...] = jnp.concatenate([hn, jnp.zeros((BR, DP - D), _f32)], axis=1)

    return pl.pallas_call(
        body,
        grid=(NB,),
        in_specs=[
            pl.BlockSpec((BR, D), lambda i: (i, 0)),
            pl.BlockSpec((1, D), lambda i: (0, 0)),
            pl.BlockSpec((1, D), lambda i: (0, 0)),
            pl.BlockSpec((1, D), lambda i: (0, 0)),
            pl.BlockSpec((1, D), lambda i: (0, 0)),
        ],
        out_specs=pl.BlockSpec((BR, DP), lambda i: (i, 0)),
        out_shape=jax.ShapeDtypeStruct((NP, DP), _f32),
    )(u, s1, s2, g, b)


def _tc_pool(h, batchp, fw, fb, ow1, ob1, ow2, ob2):
    def body(h_ref, bt_ref, fw_ref, fb_ref, ow1_ref, ob1_ref, ow2_ref,
             ob2_ref, hg_ref, og_ref):
        gi = lax.broadcasted_iota(_i32, (G, NP), 0)
        m = (gi == bt_ref[...]).astype(_f32)
        counts = jnp.sum(m, axis=1, keepdims=True)
        pooled = jnp.dot(m, h_ref[:, :D], preferred_element_type=_f32)
        pooled = pooled / jnp.maximum(counts, 1.0)
        hg = jnp.dot(pooled, fw_ref[...], preferred_element_type=_f32) + fb_ref[...]
        t = jnp.maximum(
            jnp.dot(hg, ow1_ref[...], preferred_element_type=_f32) + ob1_ref[...],
            0.0)
        og = jnp.dot(t, ow2_ref[...], preferred_element_type=_f32) + ob2_ref[...]
        hg_ref[...] = hg
        og_ref[...] = og

    return pl.pallas_call(
        body,
        out_shape=[
            jax.ShapeDtypeStruct((G, D), _f32),
            jax.ShapeDtypeStruct((G, D // 2), _f32),
        ],
    )(h, batchp, fw, fb, ow1, ob1, ow2, ob2)


_A0 = np.array([0, 0, 0, 1, 1, 1, 2, 2, 2], dtype=np.int32)
_A1 = np.array([0, 1, 2, 0, 1, 2, 0, 1, 2], dtype=np.int32)
_ITAB = np.zeros((16, 16), dtype=np.float32)
for _j in range(9):
    _ITAB[_j, _j] = 1.0


def kernel(x, edge_index, edge_attr, batch, x_emb1, x_emb2, edge_e1, edge_e2,
           W1, b1, W2, b2, bn_g, bn_b, feat_W, feat_b, out_W1, out_b1,
           out_W2, out_b2):
    src = edge_index[0].astype(_i32)
    dst = edge_index[1].astype(_i32)
    a = (edge_attr[:, 0] * 3 + edge_attr[:, 1]).astype(_i32)

    def pack(v, padval):
        vw = v.reshape(NW, EW)
        padc = jnp.full((NW, CW * K - EW), padval, _i32)
        return jnp.concatenate([vw, padc], axis=1).reshape(NW, CW, K)

    srcw = pack(src, 0)
    dstw = pack(dst, NP - 1)
    aw = pack(a, 15)

    x0 = jnp.pad(x[:, 0].astype(_i32), (0, NP - N)).reshape(NP, 1)
    x1 = jnp.pad(x[:, 1].astype(_i32), (0, NP - N)).reshape(NP, 1)
    emb1p = jnp.pad(x_emb1, ((0, 12), (0, DP - D)))
    emb2p = jnp.pad(x_emb2, ((0, 12), (0, DP - D)))
    zrows = jnp.zeros((RPT, DP), _f32)
    z16 = jnp.zeros((RPT, 16), _f32)
    itab = jnp.asarray(_ITAB)

    q = edge_e1[:, _A0, :] + edge_e2[:, _A1, :]
    qp = jnp.pad(q, ((0, 0), (0, 7), (0, 0)))
    selfc = (edge_e1[:, 4, :] + edge_e2[:, 0, :]).reshape(L, 1, D)

    c2 = _sc_hist(itab, aw, dstw, z16)
    h = _tc_embed(x0, x1, emb1p, emb2p)

    for l in range(L):
        part = _sc_scatter(h, srcw, dstw, zrows)
        u, s1, s2 = _tc_layer(part, h, c2, qp[l], selfc[l],
                              W1[l], b1[l].reshape(1, -1),
                              W2[l], b2[l].reshape(1, -1))
        h = _tc_norm(u, s1, s2, bn_g[l].reshape(1, -1),
                     bn_b[l].reshape(1, -1), relu=(l < L - 1))

    batchp = jnp.pad(batch.astype(_i32), (0, NP - N),
                     constant_values=G).reshape(1, NP)
    hg, og = _tc_pool(h, batchp, feat_W, feat_b.reshape(1, -1),
                      out_W1, out_b1.reshape(1, -1),
                      out_W2, out_b2.reshape(1, -1))
    return (hg, og, og)


# speedup vs baseline: 3.5036x; 3.5036x over previous
import functools

import numpy as np
import jax
import jax.numpy as jnp
from jax import lax
from jax.experimental import pallas as pl
from jax.experimental.pallas import tpu as pltpu
from jax.experimental.pallas import tpu_sc as plsc

N = 10000
E = 320000
D = 200
L = 5
G = 100

NC, NS = 2, 16
NP = 10048
HP = 128
K = 128
EW = E // NS
CW = (EW + K - 1) // K
NRA = 9080
ALO = 9088
NRB = NP - NRA
ZR = 568
NB = 8
BR = NP // NB

_f32 = jnp.float32
_bf16 = jnp.bfloat16
_i32 = jnp.int32


def _sc_segsum(table2, rowidx, dstidx, zrows):
    @functools.partial(
        pl.kernel,
        out_type=jax.ShapeDtypeStruct((NC, NP, HP), _f32),
        mesh=plsc.VectorSubcoreMesh(core_axis_name="c", subcore_axis_name="s"),
        scratch_types=[
            pltpu.VMEM((CW, K), _i32),
            pltpu.VMEM((CW, K), _i32),
            pltpu.VMEM((K, HP), _f32),
            pltpu.VMEM((K,), _i32),
            pltpu.VMEM_SHARED((ALO, HP), _f32),
            pltpu.SemaphoreType.DMA,
        ],
    )
    def k(t_hbm, src_hbm, dst_hbm, z_hbm, out_hbm, src_v, dst_v, buf, ib,
          acc, sem):
        cid = lax.axis_index("c")
        sid = lax.axis_index("s")
        pltpu.sync_copy(src_hbm.at[sid], src_v)
        pltpu.sync_copy(dst_hbm.at[sid], dst_v)
        plane = t_hbm.at[cid]

        def do_pass(base, nreal, spare, zero_fn, wb_fn):
            zero_fn()
            plsc.subcore_barrier()

            @pl.loop(0, CW)
            def _(c):
                pltpu.async_copy(plane.at[src_v.at[c]], buf, sem).wait()
                for kk in range(K // 16):
                    v = dst_v[c, pl.ds(kk * 16, 16)]
                    w = jnp.where((v >= base) & (v < base + nreal),
                                  v - base, spare)
                    ib[pl.ds(kk * 16, 16)] = w
                pltpu.sync_copy(buf, acc.at[ib], add=True)

            plsc.subcore_barrier()
            wb_fn()
            plsc.subcore_barrier()

        def zero_a():
            r0 = pl.multiple_of(sid * 568, 8)
            pltpu.sync_copy(z_hbm, acc.at[pl.ds(r0, 568)])

        def wb_a():
            r0 = pl.multiple_of(sid * 568, 8)
            pltpu.sync_copy(acc.at[pl.ds(r0, 568)],
                            out_hbm.at[cid, pl.ds(r0, 568)])

        def zero_b():
            @pl.when(sid < 15)
            def _():
                r0 = pl.multiple_of(sid * 64, 8)
                pltpu.sync_copy(z_hbm.at[pl.ds(0, 64)], acc.at[pl.ds(r0, 64)])

            @pl.when(sid == 15)
            def _():
                pltpu.sync_copy(z_hbm.at[pl.ds(0, 16)], acc.at[pl.ds(960, 16)])

        def wb_b():
            @pl.when(sid < 15)
            def _():
                r0 = pl.multiple_of(sid * 64, 8)
                pltpu.sync_copy(acc.at[pl.ds(r0, 64)],
                                out_hbm.at[cid, pl.ds(NRA + r0, 64)])

            @pl.when(sid == 15)
            def _():
                pltpu.sync_copy(acc.at[pl.ds(960, 8)],
                                out_hbm.at[cid, pl.ds(NRA + 960, 8)])

        do_pass(0, NRA, NRA, zero_a, wb_a)
        do_pass(NRA, NRB, NRB, zero_b, wb_b)

    return k(table2, rowidx, dstidx, zrows)


def _tc_embed(x0, x1, emb1p, emb2p):
    def body(x0_ref, x1_ref, e1_ref, e2_ref, o_ref):
        ar = lax.broadcasted_iota(_i32, (BR, 512), 1)
        m1 = (ar == x0_ref[...]).astype(_f32)
        m2 = (ar == x1_ref[...]).astype(_f32)
        h0 = (jnp.dot(m1, e1_ref[...], preferred_element_type=_f32, precision=lax.Precision.HIGHEST)
              + jnp.dot(m2, e2_ref[...], preferred_element_type=_f32, precision=lax.Precision.HIGHEST))
        o_ref[0] = h0[:, :HP]
        o_ref[1] = h0[:, HP:]

    return pl.pallas_call(
        body,
        grid=(NB,),
        in_specs=[
            pl.BlockSpec((BR, 1), lambda i: (i, 0)),
            pl.BlockSpec((BR, 1), lambda i: (i, 0)),
            pl.BlockSpec((512, 2 * HP), lambda i: (0, 0)),
            pl.BlockSpec((512, 2 * HP), lambda i: (0, 0)),
        ],
        out_specs=pl.BlockSpec((NC, BR, HP), lambda i: (0, i, 0)),
        out_shape=jax.ShapeDtypeStruct((NC, NP, HP), _f32),
    )(x0, x1, emb1p, emb2p)


def _cat_planes(p0, p1):
    return jnp.concatenate([p0, p1[:, :D - HP]], axis=1)


def _tc_layer(aggp, h2, c2, q, selfc, w1, b1, w2, b2):
    def body(p_ref, h_ref, c_ref, q_ref, sc_ref, w1_ref, b1_ref, w2_ref,
             b2_ref, u_ref, s1_ref, s2_ref):
        i = pl.program_id(0)
        agg = _cat_planes(p_ref[0], p_ref[1]) + _cat_planes(h_ref[0], h_ref[1])
        cc = c_ref[0][:, :16]
        agg = agg + jnp.dot(cc, q_ref[...], preferred_element_type=_f32, precision=lax.Precision.HIGHEST)
        agg = agg + sc_ref[...]
        t = jnp.maximum(
            jnp.dot(agg, w1_ref[...], preferred_element_type=_f32)
            + b1_ref[...], 0.0)
        u = jnp.dot(t, w2_ref[...], preferred_element_type=_f32) + b2_ref[...]
        rows = lax.broadcasted_iota(_i32, (BR, 1), 0) + i * BR
        u = jnp.where(rows < N, u, 0.0)
        u_ref[...] = u
        su = jnp.sum(u, axis=0, keepdims=True)
        sq = jnp.sum(u * u, axis=0, keepdims=True)

        @pl.when(i == 0)
        def _():
            s1_ref[...] = su
            s2_ref[...] = sq

        @pl.when(i > 0)
        def _():
            s1_ref[...] += su
            s2_ref[...] += sq

    return pl.pallas_call(
        body,
        grid=(NB,),
        in_specs=[
            pl.BlockSpec((NC, BR, HP), lambda i: (0, i, 0)),
            pl.BlockSpec((NC, BR, HP), lambda i: (0, i, 0)),
            pl.BlockSpec((NC, BR, HP), lambda i: (0, i, 0)),
            pl.BlockSpec((16, D), lambda i: (0, 0)),
            pl.BlockSpec((1, D), lambda i: (0, 0)),
            pl.BlockSpec((D, 2 * D), lambda i: (0, 0)),
            pl.BlockSpec((1, 2 * D), lambda i: (0, 0)),
            pl.BlockSpec((2 * D, D), lambda i: (0, 0)),
            pl.BlockSpec((1, D), lambda i: (0, 0)),
        ],
        out_specs=[
            pl.BlockSpec((BR, D), lambda i: (i, 0)),
            pl.BlockSpec((1, D), lambda i: (0, 0)),
            pl.BlockSpec((1, D), lambda i: (0, 0)),
        ],
        out_shape=[
            jax.ShapeDtypeStruct((NP, D), _f32),
            jax.ShapeDtypeStruct((1, D), _f32),
            jax.ShapeDtypeStruct((1, D), _f32),
        ],
    )(aggp, h2, c2, q, selfc, w1, b1, w2, b2)


def _tc_norm(u, s1, s2, g, b, relu):
    def body(u_ref, s1_ref, s2_ref, g_ref, b_ref, o_ref):
        mean = s1_ref[...] * (1.0 / N)
        var = s2_ref[...] * (1.0 / N) - mean * mean
        inv = lax.rsqrt(var + 1e-5)
        hn = (u_ref[...] - mean) * (inv * g_ref[...]) + b_ref[...]
        if relu:
            hn = jnp.maximum(hn, 0.0)
        o_ref[0] = hn[:, :HP]
        o_ref[1] = jnp.concatenate(
            [hn[:, HP:], jnp.zeros((BR, 2 * HP - D), _f32)], axis=1)

    return pl.pallas_call(
        body,
        grid=(NB,),
        in_specs=[
            pl.BlockSpec((BR, D), lambda i: (i, 0)),
            pl.BlockSpec((1, D), lambda i: (0, 0)),
            pl.BlockSpec((1, D), lambda i: (0, 0)),
            pl.BlockSpec((1, D), lambda i: (0, 0)),
            pl.BlockSpec((1, D), lambda i: (0, 0)),
        ],
        out_specs=pl.BlockSpec((NC, BR, HP), lambda i: (0, i, 0)),
        out_shape=jax.ShapeDtypeStruct((NC, NP, HP), _f32),
    )(u, s1, s2, g, b)


def _tc_pool(h2, batchp, fw, fb, ow1, ob1, ow2, ob2):
    def body(h_ref, bt_ref, fw_ref, fb_ref, ow1_ref, ob1_ref, ow2_ref,
             ob2_ref, hg_ref, og_ref):
        gi = lax.broadcasted_iota(_i32, (G, NP), 0)
        m = (gi == bt_ref[...]).astype(_f32)
        counts = jnp.sum(m, axis=1, keepdims=True)
        hfull = _cat_planes(h_ref[0], h_ref[1])
        pooled = jnp.dot(m, hfull, preferred_element_type=_f32, precision=lax.Precision.HIGHEST)
        pooled = pooled / jnp.maximum(counts, 1.0)
        hg = jnp.dot(pooled, fw_ref[...], preferred_element_type=_f32) + fb_ref[...]
        t = jnp.maximum(
            jnp.dot(hg, ow1_ref[...], preferred_element_type=_f32) + ob1_ref[...],
            0.0)
        og = jnp.dot(t, ow2_ref[...], preferred_element_type=_f32) + ob2_ref[...]
        hg_ref[...] = hg
        og_ref[...] = og

    return pl.pallas_call(
        body,
        out_shape=[
            jax.ShapeDtypeStruct((G, D), _f32),
            jax.ShapeDtypeStruct((G, D // 2), _f32),
        ],
    )(h2, batchp, fw, fb, ow1, ob1, ow2, ob2)


_A0 = np.array([0, 0, 0, 1, 1, 1, 2, 2, 2], dtype=np.int32)
_A1 = np.array([0, 1, 2, 0, 1, 2, 0, 1, 2], dtype=np.int32)
_ITAB = np.zeros((NC, 16, HP), dtype=np.float32)
for _j in range(9):
    _ITAB[:, _j, _j] = 1.0


def _pack(v, padval):
    vw = v.reshape(NS, EW)
    padc = jnp.full((NS, CW * K - EW), padval, _i32)
    return jnp.concatenate([vw, padc], axis=1).reshape(NS, CW, K)


def kernel(x, edge_index, edge_attr, batch, x_emb1, x_emb2, edge_e1, edge_e2,
           W1, b1, W2, b2, bn_g, bn_b, feat_W, feat_b, out_W1, out_b1,
           out_W2, out_b2):
    src = edge_index[0].astype(_i32)
    dst = edge_index[1].astype(_i32)
    a = (edge_attr[:, 0] * 3 + edge_attr[:, 1]).astype(_i32)

    srcw = _pack(src, 0)
    dstw = _pack(dst, NP - 1)
    aw = _pack(a, 15)

    x0 = jnp.pad(x[:, 0].astype(_i32), (0, NP - N)).reshape(NP, 1)
    x1 = jnp.pad(x[:, 1].astype(_i32), (0, NP - N)).reshape(NP, 1)
    emb1p = jnp.pad(x_emb1, ((0, 12), (0, 2 * HP - D)))
    emb2p = jnp.pad(x_emb2, ((0, 12), (0, 2 * HP - D)))
    zrows = jnp.zeros((ZR, HP), _f32)
    itab = jnp.asarray(_ITAB)

    q = edge_e1[:, _A0, :] + edge_e2[:, _A1, :]
    qp = jnp.pad(q, ((0, 0), (0, 7), (0, 0)))
    selfc = (edge_e1[:, 4, :] + edge_e2[:, 0, :]).reshape(L, 1, D)

    c2 = _sc_segsum(itab, aw, dstw, zrows)
    h = _tc_embed(x0, x1, emb1p, emb2p)

    for l in range(L):
        aggp = _sc_segsum(h, srcw, dstw, zrows)
        u, s1, s2 = _tc_layer(aggp, h, c2, qp[l], selfc[l],
                              W1[l], b1[l].reshape(1, -1),
                              W2[l], b2[l].reshape(1, -1))
        h = _tc_norm(u, s1, s2, bn_g[l].reshape(1, -1),
                     bn_b[l].reshape(1, -1), relu=(l < L - 1))

    batchp = jnp.pad(batch.astype(_i32), (0, NP - N),
                     constant_values=G).reshape(1, NP)
    hg, og = _tc_pool(h, batchp, feat_W, feat_b.reshape(1, -1),
                      out_W1, out_b1.reshape(1, -1),
                      out_W2, out_b2.reshape(1, -1))
    return (hg, og, og)


# speedup vs baseline: 3.5054x; 1.0005x over previous
import functools

import numpy as np
import jax
import jax.numpy as jnp
from jax import lax
from jax.experimental import pallas as pl
from jax.experimental.pallas import tpu as pltpu
from jax.experimental.pallas import tpu_sc as plsc

N = 10000
E = 320000
D = 200
L = 5
G = 100

NC, NS = 2, 16
NP = 10048
HP = 128
K = 128
EW = E // NS
CW = (EW + K - 1) // K
NRA = 9080
ALO = 9088
NRB = NP - NRA
ZR = 568
NB = 8
BR = NP // NB

_f32 = jnp.float32
_bf16 = jnp.bfloat16
_i32 = jnp.int32


def _sc_segsum(table2, rowidx, dstidx, zrows):
    @functools.partial(
        pl.kernel,
        out_type=jax.ShapeDtypeStruct((NC, NP, HP), _f32),
        mesh=plsc.VectorSubcoreMesh(core_axis_name="c", subcore_axis_name="s"),
        scratch_types=[
            pltpu.VMEM((CW, K), _i32),
            pltpu.VMEM((CW, K), _i32),
            pltpu.VMEM((K, HP), _f32),
            pltpu.VMEM((K,), _i32),
            pltpu.VMEM_SHARED((ALO, HP), _f32),
            pltpu.SemaphoreType.DMA,
        ],
    )
    def k(t_hbm, src_hbm, dst_hbm, z_hbm, out_hbm, src_v, dst_v, buf, ib,
          acc, sem):
        cid = lax.axis_index("c")
        sid = lax.axis_index("s")
        pltpu.sync_copy(src_hbm.at[sid], src_v)
        pltpu.sync_copy(dst_hbm.at[sid], dst_v)
        plane = t_hbm.at[cid]

        def do_pass(base, nreal, spare, zero_fn, wb_fn):
            zero_fn()
            plsc.subcore_barrier()

            @pl.loop(0, CW)
            def _(c):
                pltpu.async_copy(plane.at[src_v.at[c]], buf, sem).wait()
                for kk in range(K // 16):
                    v = dst_v[c, pl.ds(kk * 16, 16)]
                    w = jnp.where((v >= base) & (v < base + nreal),
                                  v - base, spare)
                    ib[pl.ds(kk * 16, 16)] = w
                pltpu.sync_copy(buf, acc.at[ib], add=True)

            plsc.subcore_barrier()
            wb_fn()
            plsc.subcore_barrier()

        def zero_a():
            r0 = pl.multiple_of(sid * 568, 8)
            pltpu.sync_copy(z_hbm, acc.at[pl.ds(r0, 568)])

        def wb_a():
            r0 = pl.multiple_of(sid * 568, 8)
            pltpu.sync_copy(acc.at[pl.ds(r0, 568)],
                            out_hbm.at[cid, pl.ds(r0, 568)])

        def zero_b():
            @pl.when(sid < 15)
            def _():
                r0 = pl.multiple_of(sid * 64, 8)
                pltpu.sync_copy(z_hbm.at[pl.ds(0, 64)], acc.at[pl.ds(r0, 64)])

            @pl.when(sid == 15)
            def _():
                pltpu.sync_copy(z_hbm.at[pl.ds(0, 16)], acc.at[pl.ds(960, 16)])

        def wb_b():
            @pl.when(sid < 15)
            def _():
                r0 = pl.multiple_of(sid * 64, 8)
                pltpu.sync_copy(acc.at[pl.ds(r0, 64)],
                                out_hbm.at[cid, pl.ds(NRA + r0, 64)])

            @pl.when(sid == 15)
            def _():
                pltpu.sync_copy(acc.at[pl.ds(960, 8)],
                                out_hbm.at[cid, pl.ds(NRA + 960, 8)])

        do_pass(0, NRA, NRA, zero_a, wb_a)
        do_pass(NRA, NRB, NRB, zero_b, wb_b)

    return k(table2, rowidx, dstidx, zrows)


def _tc_embed(x0, x1, emb1p, emb2p):
    def body(x0_ref, x1_ref, e1_ref, e2_ref, o_ref):
        ar = lax.broadcasted_iota(_i32, (BR, 512), 1)
        m1 = (ar == x0_ref[...]).astype(_f32)
        m2 = (ar == x1_ref[...]).astype(_f32)
        h0 = (jnp.dot(m1, e1_ref[...], preferred_element_type=_f32, precision=lax.Precision.HIGHEST)
              + jnp.dot(m2, e2_ref[...], preferred_element_type=_f32, precision=lax.Precision.HIGHEST))
        o_ref[0] = h0[:, :HP]
        o_ref[1] = h0[:, HP:]

    return pl.pallas_call(
        body,
        grid=(NB,),
        in_specs=[
            pl.BlockSpec((BR, 1), lambda i: (i, 0)),
            pl.BlockSpec((BR, 1), lambda i: (i, 0)),
            pl.BlockSpec((512, 2 * HP), lambda i: (0, 0)),
            pl.BlockSpec((512, 2 * HP), lambda i: (0, 0)),
        ],
        out_specs=pl.BlockSpec((NC, BR, HP), lambda i: (0, i, 0)),
        out_shape=jax.ShapeDtypeStruct((NC, NP, HP), _f32),
    )(x0, x1, emb1p, emb2p)


def _cat_planes(p0, p1):
    return jnp.concatenate([p0, p1[:, :D - HP]], axis=1)


def _tc_layer(aggp, h2, c2, q, selfc, w1, b1, w2, b2):
    def body(p_ref, h_ref, c_ref, q_ref, sc_ref, w1_ref, b1_ref, w2_ref,
             b2_ref, u_ref, s1_ref, s2_ref):
        i = pl.program_id(0)
        agg = _cat_planes(p_ref[0], p_ref[1]) + _cat_planes(h_ref[0], h_ref[1])
        cc = c_ref[0][:, :16]
        agg = agg + jnp.dot(cc, q_ref[...], preferred_element_type=_f32, precision=lax.Precision.HIGHEST)
        agg = agg + sc_ref[...]
        t = jnp.maximum(
            jnp.dot(agg, w1_ref[...], preferred_element_type=_f32)
            + b1_ref[...], 0.0)
        u = jnp.dot(t, w2_ref[...], preferred_element_type=_f32) + b2_ref[...]
        rows = lax.broadcasted_iota(_i32, (BR, 1), 0) + i * BR
        u = jnp.where(rows < N, u, 0.0)
        u_ref[...] = u
        su = jnp.sum(u, axis=0, keepdims=True)
        sq = jnp.sum(u * u, axis=0, keepdims=True)

        @pl.when(i == 0)
        def _():
            s1_ref[...] = su
            s2_ref[...] = sq

        @pl.when(i > 0)
        def _():
            s1_ref[...] += su
            s2_ref[...] += sq

    return pl.pallas_call(
        body,
        grid=(NB,),
        in_specs=[
            pl.BlockSpec((NC, BR, HP), lambda i: (0, i, 0)),
            pl.BlockSpec((NC, BR, HP), lambda i: (0, i, 0)),
            pl.BlockSpec((NC, BR, HP), lambda i: (0, i, 0)),
            pl.BlockSpec((16, D), lambda i: (0, 0)),
            pl.BlockSpec((1, D), lambda i: (0, 0)),
            pl.BlockSpec((D, 2 * D), lambda i: (0, 0)),
            pl.BlockSpec((1, 2 * D), lambda i: (0, 0)),
            pl.BlockSpec((2 * D, D), lambda i: (0, 0)),
            pl.BlockSpec((1, D), lambda i: (0, 0)),
        ],
        out_specs=[
            pl.BlockSpec((BR, D), lambda i: (i, 0)),
            pl.BlockSpec((1, D), lambda i: (0, 0)),
            pl.BlockSpec((1, D), lambda i: (0, 0)),
        ],
        out_shape=[
            jax.ShapeDtypeStruct((NP, D), _f32),
            jax.ShapeDtypeStruct((1, D), _f32),
            jax.ShapeDtypeStruct((1, D), _f32),
        ],
    )(aggp, h2, c2, q, selfc, w1, b1, w2, b2)


def _tc_norm(u, s1, s2, g, b, relu):
    def body(u_ref, s1_ref, s2_ref, g_ref, b_ref, o_ref):
        mean = s1_ref[...] * (1.0 / N)
        var = s2_ref[...] * (1.0 / N) - mean * mean
        hn = (u_ref[...] - mean) / jnp.sqrt(var + 1e-5) * g_ref[...] + b_ref[...]
        if relu:
            hn = jnp.maximum(hn, 0.0)
        o_ref[0] = hn[:, :HP]
        o_ref[1] = jnp.concatenate(
            [hn[:, HP:], jnp.zeros((BR, 2 * HP - D), _f32)], axis=1)

    return pl.pallas_call(
        body,
        grid=(NB,),
        in_specs=[
            pl.BlockSpec((BR, D), lambda i: (i, 0)),
            pl.BlockSpec((1, D), lambda i: (0, 0)),
            pl.BlockSpec((1, D), lambda i: (0, 0)),
            pl.BlockSpec((1, D), lambda i: (0, 0)),
            pl.BlockSpec((1, D), lambda i: (0, 0)),
        ],
        out_specs=pl.BlockSpec((NC, BR, HP), lambda i: (0, i, 0)),
        out_shape=jax.ShapeDtypeStruct((NC, NP, HP), _f32),
    )(u, s1, s2, g, b)


def _tc_pool(h2, batchp, fw, fb, ow1, ob1, ow2, ob2):
    def body(h_ref, bt_ref, fw_ref, fb_ref, ow1_ref, ob1_ref, ow2_ref,
             ob2_ref, hg_ref, og_ref):
        gi = lax.broadcasted_iota(_i32, (G, NP), 0)
        m = (gi == bt_ref[...]).astype(_f32)
        counts = jnp.sum(m, axis=1, keepdims=True)
        hfull = _cat_planes(h_ref[0], h_ref[1])
        pooled = jnp.dot(m, hfull, preferred_element_type=_f32, precision=lax.Precision.HIGHEST)
        pooled = pooled / jnp.maximum(counts, 1.0)
        hg = jnp.dot(pooled, fw_ref[...], preferred_element_type=_f32) + fb_ref[...]
        t = jnp.maximum(
            jnp.dot(hg, ow1_ref[...], preferred_element_type=_f32) + ob1_ref[...],
            0.0)
        og = jnp.dot(t, ow2_ref[...], preferred_element_type=_f32) + ob2_ref[...]
        hg_ref[...] = hg
        og_ref[...] = og

    return pl.pallas_call(
        body,
        out_shape=[
            jax.ShapeDtypeStruct((G, D), _f32),
            jax.ShapeDtypeStruct((G, D // 2), _f32),
        ],
    )(h2, batchp, fw, fb, ow1, ob1, ow2, ob2)


_A0 = np.array([0, 0, 0, 1, 1, 1, 2, 2, 2], dtype=np.int32)
_A1 = np.array([0, 1, 2, 0, 1, 2, 0, 1, 2], dtype=np.int32)
_ITAB = np.zeros((NC, 16, HP), dtype=np.float32)
for _j in range(9):
    _ITAB[:, _j, _j] = 1.0


def _pack(v, padval):
    vw = v.reshape(NS, EW)
    padc = jnp.full((NS, CW * K - EW), padval, _i32)
    return jnp.concatenate([vw, padc], axis=1).reshape(NS, CW, K)


def kernel(x, edge_index, edge_attr, batch, x_emb1, x_emb2, edge_e1, edge_e2,
           W1, b1, W2, b2, bn_g, bn_b, feat_W, feat_b, out_W1, out_b1,
           out_W2, out_b2):
    src = edge_index[0].astype(_i32)
    dst = edge_index[1].astype(_i32)
    a = (edge_attr[:, 0] * 3 + edge_attr[:, 1]).astype(_i32)

    srcw = _pack(src, 0)
    dstw = _pack(dst, NP - 1)
    aw = _pack(a, 15)

    x0 = jnp.pad(x[:, 0].astype(_i32), (0, NP - N)).reshape(NP, 1)
    x1 = jnp.pad(x[:, 1].astype(_i32), (0, NP - N)).reshape(NP, 1)
    emb1p = jnp.pad(x_emb1, ((0, 12), (0, 2 * HP - D)))
    emb2p = jnp.pad(x_emb2, ((0, 12), (0, 2 * HP - D)))
    zrows = jnp.zeros((ZR, HP), _f32)
    itab = jnp.asarray(_ITAB)

    q = edge_e1[:, _A0, :] + edge_e2[:, _A1, :]
    qp = jnp.pad(q, ((0, 0), (0, 7), (0, 0)))
    selfc = (edge_e1[:, 4, :] + edge_e2[:, 0, :]).reshape(L, 1, D)

    c2 = _sc_segsum(itab, aw, dstw, zrows)
    h = _tc_embed(x0, x1, emb1p, emb2p)

    for l in range(L):
        aggp = _sc_segsum(h, srcw, dstw, zrows)
        u, s1, s2 = _tc_layer(aggp, h, c2, qp[l], selfc[l],
                              W1[l], b1[l].reshape(1, -1),
                              W2[l], b2[l].reshape(1, -1))
        h = _tc_norm(u, s1, s2, bn_g[l].reshape(1, -1),
                     bn_b[l].reshape(1, -1), relu=(l < L - 1))

    batchp = jnp.pad(batch.astype(_i32), (0, NP - N),
                     constant_values=G).reshape(1, NP)
    hg, og = _tc_pool(h, batchp, feat_W, feat_b.reshape(1, -1),
                      out_W1, out_b1.reshape(1, -1),
                      out_W2, out_b2.reshape(1, -1))
    return (hg, og, og)
